# baseline (speedup 1.0000x reference)
import functools

import jax
import jax.numpy as jnp
from jax import lax
from jax.experimental import pallas as pl
from jax.experimental.pallas import tpu as pltpu
from jax.experimental.pallas import tpu_sc as plsc

N_RULES = 1000
D_MODEL = 512
BATCH = 1024
SEQ = 200

NW = 32
ROWS = SEQ * BATCH
ROWS_PER_W = ROWS // NW
CHUNK = 64
CHUNKS_PER_W = ROWS_PER_W // CHUNK


def _make_sc_gather():
    mesh = plsc.VectorSubcoreMesh(core_axis_name="c", subcore_axis_name="s")

    @functools.partial(
        pl.kernel,
        mesh=mesh,
        out_type=jax.ShapeDtypeStruct((ROWS, D_MODEL), jnp.float32),
        scratch_types=[
            pltpu.VMEM((CHUNKS_PER_W, CHUNK), jnp.int32),
            pltpu.VMEM((CHUNK, D_MODEL), jnp.float32),
            pltpu.VMEM_SHARED((16, 2, CHUNK, D_MODEL), jnp.float32),
            pltpu.SemaphoreType.DMA,
            pltpu.SemaphoreType.DMA,
            pltpu.SemaphoreType.DMA,
        ],
    )
    def k(table_hbm, idx_hbm, out_hbm, idx_v, rows_v, stage_sh,
          gsem, wsem0, wsem1):
        sid = lax.axis_index("s")
        wid = sid * 2 + lax.axis_index("c")
        base = wid * ROWS_PER_W
        slots = stage_sh.at[sid]

        pltpu.sync_copy(idx_hbm.at[wid], idx_v)
        wsems = (wsem0, wsem1)

        pltpu.async_copy(table_hbm.at[idx_v.at[0]], rows_v, gsem)

        def step(j2, carry):
            for s in (0, 1):
                j = j2 * 2 + s
                nxt = j + 1

                pltpu.make_async_copy(
                    table_hbm.at[idx_v.at[j]], rows_v, gsem
                ).wait()

                @pl.when(j >= 2)
                def _():
                    pltpu.make_async_copy(
                        slots.at[s], out_hbm.at[pl.ds(base, CHUNK)], wsems[s]
                    ).wait()

                pltpu.sync_copy(rows_v, slots.at[s])

                @pl.when(nxt < CHUNKS_PER_W)
                def _():
                    pltpu.async_copy(
                        table_hbm.at[idx_v.at[nxt]], rows_v, gsem
                    )

                pltpu.async_copy(
                    slots.at[s], out_hbm.at[pl.ds(base + j * CHUNK, CHUNK)],
                    wsems[s],
                )
            return carry

        lax.fori_loop(0, CHUNKS_PER_W // 2, step, 0)

        for s in (0, 1):
            pltpu.make_async_copy(
                slots.at[s], out_hbm.at[pl.ds(base, CHUNK)], wsems[s]
            ).wait()

    return k


_sc_gather = _make_sc_gather()


def kernel(states_batch, rule_embedding):
    idx_t = states_batch.T.reshape(NW, CHUNKS_PER_W, CHUNK)
    out = _sc_gather(rule_embedding, idx_t)
    return out.reshape(SEQ, BATCH, D_MODEL)


# speedup vs baseline: 1.1516x; 1.1516x over previous
import functools

import jax
import jax.numpy as jnp
from jax import lax
from jax.experimental import pallas as pl
from jax.experimental.pallas import tpu as pltpu
from jax.experimental.pallas import tpu_sc as plsc

N_RULES = 1000
D_MODEL = 512
BATCH = 1024
SEQ = 200

NW = 32
ROWS = SEQ * BATCH
ROWS_PER_W = ROWS // NW
CHUNK = 64
CHUNKS_PER_W = ROWS_PER_W // CHUNK


def _make_sc_gather():
    mesh = plsc.VectorSubcoreMesh(core_axis_name="c", subcore_axis_name="s")

    @functools.partial(
        pl.kernel,
        mesh=mesh,
        out_type=jax.ShapeDtypeStruct((ROWS, D_MODEL), jnp.float32),
        scratch_types=[
            pltpu.VMEM((CHUNKS_PER_W, CHUNK), jnp.int32),
            pltpu.VMEM((2, CHUNK, D_MODEL), jnp.float32),
            pltpu.VMEM_SHARED((16, CHUNK, D_MODEL), jnp.float32),
            pltpu.SemaphoreType.DMA,
            pltpu.SemaphoreType.DMA,
            pltpu.SemaphoreType.DMA,
        ],
    )
    def k(table_hbm, idx_hbm, out_hbm, idx_v, rows_v, stage_sh,
          gsem0, gsem1, wsem):
        sid = lax.axis_index("s")
        wid = sid * 2 + lax.axis_index("c")
        base = wid * ROWS_PER_W
        slot = stage_sh.at[sid]

        pltpu.sync_copy(idx_hbm.at[wid], idx_v)
        gsems = (gsem0, gsem1)

        pltpu.async_copy(table_hbm.at[idx_v.at[0]], rows_v.at[0], gsem0)

        def step(j2, carry):
            for s in (0, 1):
                j = j2 * 2 + s
                nxt = j + 1

                @pl.when(nxt < CHUNKS_PER_W)
                def _():
                    pltpu.async_copy(
                        table_hbm.at[idx_v.at[nxt]], rows_v.at[1 - s],
                        gsems[1 - s],
                    )

                pltpu.make_async_copy(
                    table_hbm.at[idx_v.at[j]], rows_v.at[s], gsems[s]
                ).wait()

                @pl.when(j >= 1)
                def _():
                    pltpu.make_async_copy(
                        slot, out_hbm.at[pl.ds(base, CHUNK)], wsem
                    ).wait()

                pltpu.sync_copy(rows_v.at[s], slot)
                pltpu.async_copy(
                    slot, out_hbm.at[pl.ds(base + j * CHUNK, CHUNK)], wsem
                )
            return carry

        lax.fori_loop(0, CHUNKS_PER_W // 2, step, 0)

        pltpu.make_async_copy(
            slot, out_hbm.at[pl.ds(base, CHUNK)], wsem
        ).wait()

    return k


_sc_gather = _make_sc_gather()


def kernel(states_batch, rule_embedding):
    idx_t = states_batch.T.reshape(NW, CHUNKS_PER_W, CHUNK)
    out = _sc_gather(rule_embedding, idx_t)
    return out.reshape(SEQ, BATCH, D_MODEL)


# speedup vs baseline: 1.1555x; 1.0034x over previous
import functools

import jax
import jax.numpy as jnp
from jax import lax
from jax.experimental import pallas as pl
from jax.experimental.pallas import tpu as pltpu
from jax.experimental.pallas import tpu_sc as plsc

N_RULES = 1000
D_MODEL = 512
BATCH = 1024
SEQ = 200

NW = 32
ROWS = SEQ * BATCH
ROWS_PER_W = ROWS // NW
CHUNK = 64
HALF = CHUNK // 2
CHUNKS_PER_W = ROWS_PER_W // CHUNK


def _make_sc_gather():
    mesh = plsc.VectorSubcoreMesh(core_axis_name="c", subcore_axis_name="s")

    @functools.partial(
        pl.kernel,
        mesh=mesh,
        out_type=jax.ShapeDtypeStruct((ROWS, D_MODEL), jnp.float32),
        scratch_types=[
            pltpu.VMEM((CHUNKS_PER_W, CHUNK), jnp.int32),
            pltpu.VMEM((2, CHUNK, D_MODEL), jnp.float32),
            pltpu.VMEM_SHARED((16, 2, HALF, D_MODEL), jnp.float32),
            pltpu.SemaphoreType.DMA,
            pltpu.SemaphoreType.DMA,
            pltpu.SemaphoreType.DMA,
            pltpu.SemaphoreType.DMA,
        ],
    )
    def k(table_hbm, idx_hbm, out_hbm, idx_v, rows_v, stage_sh,
          gsem0, gsem1, wsem0, wsem1):
        sid = lax.axis_index("s")
        wid = sid * 2 + lax.axis_index("c")
        base = wid * ROWS_PER_W
        slots = stage_sh.at[sid]

        pltpu.sync_copy(idx_hbm.at[wid], idx_v)
        gsems = (gsem0, gsem1)
        wsems = (wsem0, wsem1)

        pltpu.async_copy(table_hbm.at[idx_v.at[0]], rows_v.at[0], gsem0)

        def step(j2, carry):
            for s in (0, 1):
                j = j2 * 2 + s
                nxt = j + 1

                @pl.when(nxt < CHUNKS_PER_W)
                def _():
                    pltpu.async_copy(
                        table_hbm.at[idx_v.at[nxt]], rows_v.at[1 - s],
                        gsems[1 - s],
                    )

                pltpu.make_async_copy(
                    table_hbm.at[idx_v.at[j]], rows_v.at[s], gsems[s]
                ).wait()

                for h in (0, 1):
                    @pl.when(j >= 1)
                    def _():
                        pltpu.make_async_copy(
                            slots.at[h], out_hbm.at[pl.ds(base, HALF)],
                            wsems[h],
                        ).wait()

                    pltpu.sync_copy(
                        rows_v.at[s].at[pl.ds(h * HALF, HALF)], slots.at[h]
                    )
                    pltpu.async_copy(
                        slots.at[h],
                        out_hbm.at[pl.ds(base + j * CHUNK + h * HALF, HALF)],
                        wsems[h],
                    )
            return carry

        lax.fori_loop(0, CHUNKS_PER_W // 2, step, 0)

        for h in (0, 1):
            pltpu.make_async_copy(
                slots.at[h], out_hbm.at[pl.ds(base, HALF)], wsems[h]
            ).wait()

    return k


_sc_gather = _make_sc_gather()


def kernel(states_batch, rule_embedding):
    idx_t = states_batch.T.reshape(NW, CHUNKS_PER_W, CHUNK)
    out = _sc_gather(rule_embedding, idx_t)
    return out.reshape(SEQ, BATCH, D_MODEL)
